# baseline (speedup 1.0000x reference)
import functools

import jax
import jax.numpy as jnp
import numpy as np
from jax import lax
from jax.experimental import pallas as pl
from jax.experimental.pallas import tpu as pltpu
from jax.experimental.pallas import tpu_sc as plsc

_SIZES = (18, 17, 13, 13, 13, 11, 4)
_OFFS = tuple(int(x) for x in np.cumsum((0,) + _SIZES))
_NK = 7
_H = 256
_D = 768
_TW = 128
_BBLK = 2048
_B = 16384
_NW = 32
_BPW = _B // _NW
_CH = 32
_NCH = _BPW // _CH


def _fold_body(i0, i1, i2, i3, i4, i5, i6, e0, e1, e2, e3, e4, e5, e6,
               w1_ref, m_ref, cidx_ref):
    embcat = jnp.concatenate(
        [e[...] for e in (e0, e1, e2, e3, e4, e5, e6)]
        + [jnp.zeros((_TW - _OFFS[-1], _H), jnp.float32)], axis=0)
    riota = jax.lax.broadcasted_iota(jnp.int32, (_TW, _H), 0)
    acc = jnp.zeros((_TW, _H), jnp.float32)
    for k in range(_NK):
        mask = (riota >= _OFFS[k]) & (riota < _OFFS[k + 1])
        ek = jnp.where(mask, embcat, 0.0)
        acc += jnp.dot(ek, w1_ref[k * _H:(k + 1) * _H, :],
                       preferred_element_type=jnp.float32)
    m_ref[...] = acc
    ids = [r[...].astype(jnp.int32) + _OFFS[k]
           for k, r in enumerate((i0, i1, i2, i3, i4, i5, i6))]
    cidx_ref[...] = jnp.stack(ids + [jnp.zeros((_B,), jnp.int32)], axis=0)


@jax.jit
def _fold(idxs, embs, W1):
    idx_specs = [pl.BlockSpec((_B,), lambda i: (0,)) for _ in range(_NK)]
    emb_specs = [pl.BlockSpec((_SIZES[k], _H), lambda i: (0, 0))
                 for k in range(_NK)]
    return pl.pallas_call(
        _fold_body,
        grid=(1,),
        in_specs=idx_specs + emb_specs + [
            pl.BlockSpec((_H * _NK, _H), lambda i: (0, 0)),
        ],
        out_specs=[pl.BlockSpec((_TW, _H), lambda i: (0, 0)),
                   pl.BlockSpec((8, _B), lambda i: (0, 0))],
        out_shape=[jax.ShapeDtypeStruct((_TW, _H), jnp.float32),
                   jax.ShapeDtypeStruct((8, _B), jnp.int32)],
    )(*idxs, *embs, W1)


@functools.partial(
    pl.kernel,
    out_type=jax.ShapeDtypeStruct((_B, _H), jnp.float32),
    mesh=plsc.VectorSubcoreMesh(core_axis_name="c", subcore_axis_name="s"),
    scratch_types=[pltpu.VMEM((_CH,), jnp.int32)]
    + [pltpu.VMEM((_CH, _H), jnp.float32) for _ in range(_NK)]
    + [pltpu.VMEM((_CH, _H), jnp.float32), pltpu.SemaphoreType.DMA],
)
def _sc_gather(cidx_hbm, m_hbm, out_hbm, idxv,
               r0, r1, r2, r3, r4, r5, r6, outv, sem):
    rows = (r0, r1, r2, r3, r4, r5, r6)
    wid = lax.axis_index("s") * 2 + lax.axis_index("c")
    base = wid * _BPW

    def chunk_body(c, carry):
        cb = base + c * _CH
        for k in range(_NK):
            pltpu.sync_copy(cidx_hbm.at[k, pl.ds(cb, _CH)], idxv)
            pltpu.async_copy(m_hbm.at[idxv], rows[k], sem).wait()

        def row_body(i, carry2):
            for l in range(_H // 16):
                s = pl.ds(l * 16, 16)
                v = rows[0][i, s]
                for k in range(1, _NK):
                    v = v + rows[k][i, s]
                outv[i, s] = v
            return carry2

        lax.fori_loop(0, _CH, row_body, 0)
        pltpu.sync_copy(outv, out_hbm.at[pl.ds(cb, _CH)])
        return carry

    lax.fori_loop(0, _NCH, chunk_body, 0)


def _mlp2_body(h_ref, b1_ref, w2_ref, b2_ref, out_ref, w2b_ref):
    @pl.when(pl.program_id(0) == 0)
    def _():
        w2b_ref[...] = w2_ref[...].astype(jnp.bfloat16)

    h = jnp.maximum(h_ref[...] + b1_ref[...], 0.0).astype(jnp.bfloat16)
    out_ref[...] = jnp.dot(h, w2b_ref[...],
                           preferred_element_type=jnp.float32) + b2_ref[...]


@jax.jit
def _mlp2(h, b1, W2, b2):
    return pl.pallas_call(
        _mlp2_body,
        grid=(_B // _BBLK,),
        in_specs=[
            pl.BlockSpec((_BBLK, _H), lambda i: (i, 0)),
            pl.BlockSpec((1, _H), lambda i: (0, 0)),
            pl.BlockSpec((_H, _D), lambda i: (0, 0)),
            pl.BlockSpec((1, _D), lambda i: (0, 0)),
        ],
        out_specs=pl.BlockSpec((_BBLK, _D), lambda i: (i, 0)),
        out_shape=jax.ShapeDtypeStruct((_B, _D), jnp.float32),
        scratch_shapes=[pltpu.VMEM((_H, _D), jnp.bfloat16)],
        compiler_params=pltpu.CompilerParams(
            dimension_semantics=("arbitrary",)),
    )(h, b1, W2, b2)


def kernel(idx_primary_color, idx_secondary_color, idx_primary_material,
           idx_secondary_material, idx_style, idx_shape, idx_assembly,
           emb_primary_color, emb_secondary_color, emb_primary_material,
           emb_secondary_material, emb_style, emb_shape, emb_assembly,
           W1, b1, W2, b2):
    idxs = [idx_primary_color.astype(jnp.int32),
            idx_secondary_color.astype(jnp.int32),
            idx_primary_material.astype(jnp.int32),
            idx_secondary_material.astype(jnp.int32),
            idx_style.astype(jnp.int32),
            idx_shape.astype(jnp.int32),
            idx_assembly.astype(jnp.int32)]
    embs = [emb_primary_color, emb_secondary_color, emb_primary_material,
            emb_secondary_material, emb_style, emb_shape, emb_assembly]
    m, cidx = _fold(idxs, embs, W1)
    h_pre = _sc_gather(cidx, m)
    return _mlp2(h_pre, b1.reshape(1, _H), W2, b2.reshape(1, _D))


# speedup vs baseline: 1.1138x; 1.1138x over previous
import functools

import jax
import jax.numpy as jnp
import numpy as np
from jax import lax
from jax.experimental import pallas as pl
from jax.experimental.pallas import tpu as pltpu
from jax.experimental.pallas import tpu_sc as plsc

_SIZES = (18, 17, 13, 13, 13, 11, 4)
_OFFS = tuple(int(x) for x in np.cumsum((0,) + _SIZES))
_NK = 7
_H = 256
_D = 768
_TW = 128
_BBLK = 2048
_B = 16384
_NW = 32
_BPW = _B // _NW
_CH = 64
_NCH = _BPW // _CH


def _fold_body(i0, i1, i2, i3, i4, i5, i6, e0, e1, e2, e3, e4, e5, e6,
               w1_ref, m_ref, cidx_ref):
    embcat = jnp.concatenate(
        [e[...] for e in (e0, e1, e2, e3, e4, e5, e6)]
        + [jnp.zeros((_TW - _OFFS[-1], _H), jnp.float32)], axis=0)
    riota = jax.lax.broadcasted_iota(jnp.int32, (_TW, _H), 0)
    acc = jnp.zeros((_TW, _H), jnp.float32)
    for k in range(_NK):
        mask = (riota >= _OFFS[k]) & (riota < _OFFS[k + 1])
        ek = jnp.where(mask, embcat, 0.0)
        acc += jnp.dot(ek, w1_ref[k * _H:(k + 1) * _H, :],
                       preferred_element_type=jnp.float32)
    m_ref[...] = acc
    ids = [r[...].astype(jnp.int32) + _OFFS[k]
           for k, r in enumerate((i0, i1, i2, i3, i4, i5, i6))]
    cidx_ref[...] = jnp.stack(ids + [jnp.zeros((_B,), jnp.int32)], axis=0)


@jax.jit
def _fold(idxs, embs, W1):
    idx_specs = [pl.BlockSpec((_B,), lambda i: (0,)) for _ in range(_NK)]
    emb_specs = [pl.BlockSpec((_SIZES[k], _H), lambda i: (0, 0))
                 for k in range(_NK)]
    return pl.pallas_call(
        _fold_body,
        grid=(1,),
        in_specs=idx_specs + emb_specs + [
            pl.BlockSpec((_H * _NK, _H), lambda i: (0, 0)),
        ],
        out_specs=[pl.BlockSpec((_TW, _H), lambda i: (0, 0)),
                   pl.BlockSpec((8, _B), lambda i: (0, 0))],
        out_shape=[jax.ShapeDtypeStruct((_TW, _H), jnp.float32),
                   jax.ShapeDtypeStruct((8, _B), jnp.int32)],
    )(*idxs, *embs, W1)


@functools.partial(
    pl.kernel,
    out_type=jax.ShapeDtypeStruct((_B, _H), jnp.float32),
    mesh=plsc.VectorSubcoreMesh(core_axis_name="c", subcore_axis_name="s"),
    scratch_types=[pltpu.VMEM((8, _BPW), jnp.int32)]
    + [pltpu.VMEM((_CH, _H), jnp.float32) for _ in range(_NK)]
    + [pltpu.SemaphoreType.DMA],
)
def _sc_gather(cidx_hbm, m_hbm, out_hbm, idxv,
               r0, r1, r2, r3, r4, r5, r6, sem):
    rows = (r0, r1, r2, r3, r4, r5, r6)
    wid = lax.axis_index("s") * 2 + lax.axis_index("c")
    base = wid * _BPW
    pltpu.sync_copy(cidx_hbm.at[:, pl.ds(base, _BPW)], idxv)

    def chunk_body(c, carry):
        cb = c * _CH
        copies = [pltpu.async_copy(m_hbm.at[idxv.at[k, pl.ds(cb, _CH)]],
                                   rows[k], sem)
                  for k in range(_NK)]
        for cp in copies:
            cp.wait()

        def row_body(i, carry2):
            for l in range(_H // 16):
                s = pl.ds(l * 16, 16)
                v = rows[0][i, s]
                for k in range(1, _NK):
                    v = v + rows[k][i, s]
                rows[0][i, s] = v
            return carry2

        lax.fori_loop(0, _CH, row_body, 0)
        pltpu.sync_copy(rows[0], out_hbm.at[pl.ds(base + cb, _CH)])
        return carry

    lax.fori_loop(0, _NCH, chunk_body, 0)


def _mlp2_body(h_ref, b1_ref, w2_ref, b2_ref, out_ref, w2b_ref):
    @pl.when(pl.program_id(0) == 0)
    def _():
        w2b_ref[...] = w2_ref[...].astype(jnp.bfloat16)

    h = jnp.maximum(h_ref[...] + b1_ref[...], 0.0).astype(jnp.bfloat16)
    out_ref[...] = jnp.dot(h, w2b_ref[...],
                           preferred_element_type=jnp.float32) + b2_ref[...]


@jax.jit
def _mlp2(h, b1, W2, b2):
    return pl.pallas_call(
        _mlp2_body,
        grid=(_B // _BBLK,),
        in_specs=[
            pl.BlockSpec((_BBLK, _H), lambda i: (i, 0)),
            pl.BlockSpec((1, _H), lambda i: (0, 0)),
            pl.BlockSpec((_H, _D), lambda i: (0, 0)),
            pl.BlockSpec((1, _D), lambda i: (0, 0)),
        ],
        out_specs=pl.BlockSpec((_BBLK, _D), lambda i: (i, 0)),
        out_shape=jax.ShapeDtypeStruct((_B, _D), jnp.float32),
        scratch_shapes=[pltpu.VMEM((_H, _D), jnp.bfloat16)],
        compiler_params=pltpu.CompilerParams(
            dimension_semantics=("arbitrary",)),
    )(h, b1, W2, b2)


def kernel(idx_primary_color, idx_secondary_color, idx_primary_material,
           idx_secondary_material, idx_style, idx_shape, idx_assembly,
           emb_primary_color, emb_secondary_color, emb_primary_material,
           emb_secondary_material, emb_style, emb_shape, emb_assembly,
           W1, b1, W2, b2):
    idxs = [idx_primary_color.astype(jnp.int32),
            idx_secondary_color.astype(jnp.int32),
            idx_primary_material.astype(jnp.int32),
            idx_secondary_material.astype(jnp.int32),
            idx_style.astype(jnp.int32),
            idx_shape.astype(jnp.int32),
            idx_assembly.astype(jnp.int32)]
    embs = [emb_primary_color, emb_secondary_color, emb_primary_material,
            emb_secondary_material, emb_style, emb_shape, emb_assembly]
    m, cidx = _fold(idxs, embs, W1)
    h_pre = _sc_gather(cidx, m)
    return _mlp2(h_pre, b1.reshape(1, _H), W2, b2.reshape(1, _D))


# speedup vs baseline: 14.1147x; 12.6725x over previous
import jax
import jax.numpy as jnp
import numpy as np
from jax.experimental import pallas as pl
from jax.experimental.pallas import tpu as pltpu

_SIZES = (18, 17, 13, 13, 13, 11, 4)
_OFFS = tuple(int(x) for x in np.cumsum((0,) + _SIZES))
_NK = 7
_H = 256
_D = 768
_TW = 128
_BBLK = 2048

_PAUG = np.zeros((8, _TW), np.float32)
for _k in range(_NK):
    _PAUG[_k, _OFFS[_k]:_OFFS[_k + 1]] = 1.0
    _PAUG[7, _OFFS[_k]:_OFFS[_k + 1]] = _OFFS[_k]


def _body(i0, i1, i2, i3, i4, i5, i6, e0, e1, e2, e3, e4, e5, e6,
          w1_ref, b1_ref, w2_ref, b2_ref, paug_ref,
          out_ref, m_ref, w2b_ref):
    @pl.when(pl.program_id(0) == 0)
    def _():
        embcat = jnp.concatenate(
            [e[...] for e in (e0, e1, e2, e3, e4, e5, e6)]
            + [jnp.zeros((_TW - _OFFS[-1], _H), jnp.float32)], axis=0)
        riota = jax.lax.broadcasted_iota(jnp.int32, (_TW, _H), 0)
        acc = jnp.zeros((_TW, _H), jnp.float32)
        for k in range(_NK):
            mask = (riota >= _OFFS[k]) & (riota < _OFFS[k + 1])
            ek = jnp.where(mask, embcat, 0.0)
            acc += jnp.dot(ek, w1_ref[k * _H:(k + 1) * _H, :],
                           preferred_element_type=jnp.float32)
        m_ref[...] = acc.astype(jnp.bfloat16)
        w2b_ref[...] = w2_ref[...].astype(jnp.bfloat16)

    bblk = i0.shape[0]
    ids8 = jnp.stack(
        [r[...].astype(jnp.float32) for r in (i0, i1, i2, i3, i4, i5, i6)]
        + [jnp.ones((bblk,), jnp.float32)], axis=0)
    ids_t = ids8.T
    c = jnp.dot(ids_t, paug_ref[...],
                preferred_element_type=jnp.float32).astype(jnp.int32)
    iota = jax.lax.broadcasted_iota(jnp.int32, (bblk, _TW), 1)
    mh = (c == iota).astype(jnp.bfloat16)
    h = jnp.dot(mh, m_ref[...], preferred_element_type=jnp.float32)
    h = jnp.maximum(h + b1_ref[...], 0.0).astype(jnp.bfloat16)
    out_ref[...] = jnp.dot(h, w2b_ref[...],
                           preferred_element_type=jnp.float32) + b2_ref[...]


@jax.jit
def _run(idxs, embs, W1, b1, W2, b2):
    B = idxs[0].shape[0]
    grid = B // _BBLK
    idx_specs = [pl.BlockSpec((_BBLK,), lambda i: (i,)) for _ in range(_NK)]
    emb_specs = [pl.BlockSpec((_SIZES[k], _H), lambda i: (0, 0))
                 for k in range(_NK)]
    return pl.pallas_call(
        _body,
        grid=(grid,),
        in_specs=idx_specs + emb_specs + [
            pl.BlockSpec((_H * _NK, _H), lambda i: (0, 0)),
            pl.BlockSpec((1, _H), lambda i: (0, 0)),
            pl.BlockSpec((_H, _D), lambda i: (0, 0)),
            pl.BlockSpec((1, _D), lambda i: (0, 0)),
            pl.BlockSpec((8, _TW), lambda i: (0, 0)),
        ],
        out_specs=pl.BlockSpec((_BBLK, _D), lambda i: (i, 0)),
        out_shape=jax.ShapeDtypeStruct((B, _D), jnp.float32),
        scratch_shapes=[pltpu.VMEM((_TW, _H), jnp.bfloat16),
                        pltpu.VMEM((_H, _D), jnp.bfloat16)],
        compiler_params=pltpu.CompilerParams(
            dimension_semantics=("arbitrary",)),
    )(*idxs, *embs, W1, b1, W2, b2, jnp.asarray(_PAUG))


def kernel(idx_primary_color, idx_secondary_color, idx_primary_material,
           idx_secondary_material, idx_style, idx_shape, idx_assembly,
           emb_primary_color, emb_secondary_color, emb_primary_material,
           emb_secondary_material, emb_style, emb_shape, emb_assembly,
           W1, b1, W2, b2):
    idxs = [idx_primary_color.astype(jnp.int32),
            idx_secondary_color.astype(jnp.int32),
            idx_primary_material.astype(jnp.int32),
            idx_secondary_material.astype(jnp.int32),
            idx_style.astype(jnp.int32),
            idx_shape.astype(jnp.int32),
            idx_assembly.astype(jnp.int32)]
    embs = [emb_primary_color, emb_secondary_color, emb_primary_material,
            emb_secondary_material, emb_style, emb_shape, emb_assembly]
    return _run(idxs, embs, W1, b1.reshape(1, _H), W2, b2.reshape(1, _D))
